# baseline (speedup 1.0000x reference)
import jax
import jax.numpy as jnp
from jax.experimental import pallas as pl
from jax.experimental.pallas import tpu as pltpu

N = 1_000_000
TAU = 0.9

BLK = 655360
ROWS = BLK // 128
GRID = (N + BLK - 1) // BLK
TAIL = N - (GRID - 1) * BLK

FBLK = 1024
FROWS = FBLK // 128


def _local_iota():
    r = jax.lax.broadcasted_iota(jnp.int32, (ROWS, 128), 0)
    c = jax.lax.broadcasted_iota(jnp.int32, (ROWS, 128), 1)
    return r * 128 + c


def _main_body(sc_ref, vm_ref, wl_ref, ls_ref,
               v_out, wl_out, ls_out, widx_out, mx, best, bidx):
    i = pl.program_id(0)

    @pl.when(i == 0)
    def _():
        mx[0] = -jnp.inf

    @pl.when(i < GRID - 1)
    def _():
        mx[0] = jnp.maximum(mx[0], jnp.max(wl_ref[...]))

    @pl.when(i == GRID - 1)
    def _():
        x = wl_ref[...].reshape(ROWS, 128)
        bmax = jnp.max(jnp.where(_local_iota() < TAIL, x, -jnp.inf))
        mx[0] = jnp.maximum(mx[0], bmax)
        best[0] = -jnp.inf
        bidx[0] = 0

    @pl.when(i >= GRID)
    def _():
        denom = mx[0] + 1e-06
        ic = sc_ref[0]
        tsf = sc_ref[1]

        vm = vm_ref[...].reshape(ROWS, 128)
        wl = wl_ref[...].reshape(ROWS, 128)
        ls = ls_ref[...].reshape(ROWS, 128)

        v = TAU * vm + (1.0 - wl / denom)
        v = v + ic * (1.0 / (wl + 0.1))
        v = v + 0.1 * jnp.log1p(tsf - ls)

        v_out[...] = v.reshape(BLK)
        wl_out[...] = wl_ref[...]
        ls_out[...] = ls_ref[...]

        li = _local_iota()

        @pl.when(i < 2 * GRID - 1)
        def _():
            bmax = jnp.max(v)
            cand = jnp.min(jnp.where(v == bmax, li, jnp.int32(N)))
            pred = bmax > best[0]
            bidx[0] = jnp.where(pred, (i - GRID) * BLK + cand, bidx[0])
            best[0] = jnp.where(pred, bmax, best[0])

        @pl.when(i == 2 * GRID - 1)
        def _():
            masked = jnp.where(li < TAIL, v, -jnp.inf)
            bmax = jnp.max(masked)
            cand = jnp.min(jnp.where(masked == bmax, li, jnp.int32(N)))
            pred = bmax > best[0]
            widx_out[0] = jnp.where(pred, (i - GRID) * BLK + cand, bidx[0])


def _fix_body(w_ref, sc_ref, v_ref, wl_ref, ls_ref, vo_ref, wlo_ref, lso_ref):
    off = w_ref[0] % FBLK
    add = sc_ref[2]
    tsf = sc_ref[1]
    r = jax.lax.broadcasted_iota(jnp.int32, (FROWS, 128), 0)
    c = jax.lax.broadcasted_iota(jnp.int32, (FROWS, 128), 1)
    hit = (r * 128 + c) == off
    v = v_ref[...].reshape(FROWS, 128)
    wl = wl_ref[...].reshape(FROWS, 128)
    ls = ls_ref[...].reshape(FROWS, 128)
    vo_ref[...] = jnp.where(hit, 0.0, v).reshape(FBLK)
    wlo_ref[...] = jnp.where(hit, wl + add, wl).reshape(FBLK)
    lso_ref[...] = jnp.where(hit, tsf, ls).reshape(FBLK)


def kernel(v_mem, worker_load, last_spike, task_priority, task_complexity, timestep):
    f32 = jnp.float32
    tsf = f32(timestep)
    ic = task_priority * (1.0 + task_complexity)
    sc = jnp.stack([ic, tsf, task_complexity])

    wl_ix = lambda i: (jnp.where(i < GRID, i, i - GRID),)
    b_ix = lambda i: (jnp.maximum(i - GRID, 0),)
    smem = pl.BlockSpec(memory_space=pltpu.SMEM)
    v, wl_c, ls_c, widx = pl.pallas_call(
        _main_body,
        grid=(2 * GRID,),
        in_specs=[smem,
                  pl.BlockSpec((BLK,), b_ix),
                  pl.BlockSpec((BLK,), wl_ix),
                  pl.BlockSpec((BLK,), b_ix)],
        out_specs=[pl.BlockSpec((BLK,), b_ix)] * 3 + [smem],
        out_shape=[
            jax.ShapeDtypeStruct((N,), f32),
            jax.ShapeDtypeStruct((N,), f32),
            jax.ShapeDtypeStruct((N,), f32),
            jax.ShapeDtypeStruct((1,), jnp.int32),
        ],
        scratch_shapes=[pltpu.SMEM((1,), f32), pltpu.SMEM((1,), f32),
                        pltpu.SMEM((1,), jnp.int32)],
        compiler_params=pltpu.CompilerParams(
            dimension_semantics=("arbitrary",)),
    )(sc, v_mem, worker_load, last_spike)

    fblk = pl.BlockSpec((FBLK,), lambda i, w: (w[0] // FBLK,))
    grid_spec = pltpu.PrefetchScalarGridSpec(
        num_scalar_prefetch=1,
        grid=(1,),
        in_specs=[smem, fblk, fblk, fblk],
        out_specs=[fblk, fblk, fblk],
    )
    v_new, wl_new, ls_new = pl.pallas_call(
        _fix_body,
        grid_spec=grid_spec,
        out_shape=[jax.ShapeDtypeStruct((N,), f32)] * 3,
        input_output_aliases={2: 0, 3: 1, 4: 2},
    )(widx, sc, v, wl_c, ls_c)

    return widx[0], v_new, wl_new, ls_new


# speedup vs baseline: 1.1040x; 1.1040x over previous
import jax
import jax.numpy as jnp
from jax.experimental import pallas as pl
from jax.experimental.pallas import tpu as pltpu

N = 1_000_000
TAU = 0.9

BLK = 512000
ROWS = BLK // 128
GRID = (N + BLK - 1) // BLK
TAIL = N - (GRID - 1) * BLK

FBLK = 1024
FROWS = FBLK // 128


def _local_iota():
    r = jax.lax.broadcasted_iota(jnp.int32, (ROWS, 128), 0)
    c = jax.lax.broadcasted_iota(jnp.int32, (ROWS, 128), 1)
    return r * 128 + c


def _main_body(sc_ref, vm_ref, wl_ref, ls_ref,
               v_out, wl_out, ls_out, widx_out, mx, best, bidx):
    i = pl.program_id(0)

    @pl.when(i == 0)
    def _():
        mx[0] = -jnp.inf

    @pl.when(i < GRID - 1)
    def _():
        mx[0] = jnp.maximum(mx[0], jnp.max(wl_ref[...]))

    @pl.when(i == GRID - 1)
    def _():
        x = wl_ref[...].reshape(ROWS, 128)
        bmax = jnp.max(jnp.where(_local_iota() < TAIL, x, -jnp.inf))
        mx[0] = jnp.maximum(mx[0], bmax)
        best[0] = -jnp.inf
        bidx[0] = 0

    @pl.when(i >= GRID)
    def _():
        denom = mx[0] + 1e-06
        ic = sc_ref[0]
        tsf = sc_ref[1]

        vm = vm_ref[...].reshape(ROWS, 128)
        wl = wl_ref[...].reshape(ROWS, 128)
        ls = ls_ref[...].reshape(ROWS, 128)

        v = TAU * vm + (1.0 - wl / denom)
        v = v + ic * (1.0 / (wl + 0.1))
        v = v + 0.1 * jnp.log1p(tsf - ls)

        v_out[...] = v.reshape(BLK)
        wl_out[...] = wl_ref[...]
        ls_out[...] = ls_ref[...]

        li = _local_iota()

        @pl.when(i < 2 * GRID - 1)
        def _():
            bmax = jnp.max(v)
            cand = jnp.min(jnp.where(v == bmax, li, jnp.int32(N)))
            pred = bmax > best[0]
            bidx[0] = jnp.where(pred, (i - GRID) * BLK + cand, bidx[0])
            best[0] = jnp.where(pred, bmax, best[0])

        @pl.when(i == 2 * GRID - 1)
        def _():
            masked = jnp.where(li < TAIL, v, -jnp.inf)
            bmax = jnp.max(masked)
            cand = jnp.min(jnp.where(masked == bmax, li, jnp.int32(N)))
            pred = bmax > best[0]
            widx_out[0] = jnp.where(pred, (i - GRID) * BLK + cand, bidx[0])


def _fix_body(w_ref, sc_ref, v_ref, wl_ref, ls_ref, vo_ref, wlo_ref, lso_ref):
    off = w_ref[0] % FBLK
    add = sc_ref[2]
    tsf = sc_ref[1]
    r = jax.lax.broadcasted_iota(jnp.int32, (FROWS, 128), 0)
    c = jax.lax.broadcasted_iota(jnp.int32, (FROWS, 128), 1)
    hit = (r * 128 + c) == off
    v = v_ref[...].reshape(FROWS, 128)
    wl = wl_ref[...].reshape(FROWS, 128)
    ls = ls_ref[...].reshape(FROWS, 128)
    vo_ref[...] = jnp.where(hit, 0.0, v).reshape(FBLK)
    wlo_ref[...] = jnp.where(hit, wl + add, wl).reshape(FBLK)
    lso_ref[...] = jnp.where(hit, tsf, ls).reshape(FBLK)


def kernel(v_mem, worker_load, last_spike, task_priority, task_complexity, timestep):
    f32 = jnp.float32
    tsf = f32(timestep)
    ic = task_priority * (1.0 + task_complexity)
    sc = jnp.stack([ic, tsf, task_complexity])

    wl_ix = lambda i: (jnp.where(i < GRID, i, i - GRID),)
    b_ix = lambda i: (jnp.maximum(i - GRID, 0),)
    smem = pl.BlockSpec(memory_space=pltpu.SMEM)
    v, wl_c, ls_c, widx = pl.pallas_call(
        _main_body,
        grid=(2 * GRID,),
        in_specs=[smem,
                  pl.BlockSpec((BLK,), b_ix),
                  pl.BlockSpec((BLK,), wl_ix),
                  pl.BlockSpec((BLK,), b_ix)],
        out_specs=[pl.BlockSpec((BLK,), b_ix)] * 3 + [smem],
        out_shape=[
            jax.ShapeDtypeStruct((N,), f32),
            jax.ShapeDtypeStruct((N,), f32),
            jax.ShapeDtypeStruct((N,), f32),
            jax.ShapeDtypeStruct((1,), jnp.int32),
        ],
        scratch_shapes=[pltpu.SMEM((1,), f32), pltpu.SMEM((1,), f32),
                        pltpu.SMEM((1,), jnp.int32)],
        compiler_params=pltpu.CompilerParams(
            dimension_semantics=("arbitrary",)),
    )(sc, v_mem, worker_load, last_spike)

    fblk = pl.BlockSpec((FBLK,), lambda i, w: (w[0] // FBLK,))
    grid_spec = pltpu.PrefetchScalarGridSpec(
        num_scalar_prefetch=1,
        grid=(1,),
        in_specs=[smem, fblk, fblk, fblk],
        out_specs=[fblk, fblk, fblk],
    )
    v_new, wl_new, ls_new = pl.pallas_call(
        _fix_body,
        grid_spec=grid_spec,
        out_shape=[jax.ShapeDtypeStruct((N,), f32)] * 3,
        input_output_aliases={2: 0, 3: 1, 4: 2},
    )(widx, sc, v, wl_c, ls_c)

    return widx[0], v_new, wl_new, ls_new


# speedup vs baseline: 1.1174x; 1.0121x over previous
import jax
import jax.numpy as jnp
from jax.experimental import pallas as pl
from jax.experimental.pallas import tpu as pltpu

N = 1_000_000
TAU = 0.9

BLK = 500736
ROWS = BLK // 128
GRID = (N + BLK - 1) // BLK
TAIL = N - (GRID - 1) * BLK

FBLK = 1024
FROWS = FBLK // 128


def _local_iota():
    r = jax.lax.broadcasted_iota(jnp.int32, (ROWS, 128), 0)
    c = jax.lax.broadcasted_iota(jnp.int32, (ROWS, 128), 1)
    return r * 128 + c


def _main_body(sc_ref, vm_ref, wl_ref, ls_ref,
               v_out, wl_out, ls_out, widx_out, mx, best, bidx):
    i = pl.program_id(0)

    @pl.when(i == 0)
    def _():
        mx[0] = -jnp.inf

    @pl.when(i < GRID - 1)
    def _():
        mx[0] = jnp.maximum(mx[0], jnp.max(wl_ref[...]))

    @pl.when(i == GRID - 1)
    def _():
        x = wl_ref[...].reshape(ROWS, 128)
        bmax = jnp.max(jnp.where(_local_iota() < TAIL, x, -jnp.inf))
        mx[0] = jnp.maximum(mx[0], bmax)
        best[0] = -jnp.inf
        bidx[0] = 0

    @pl.when(i >= GRID)
    def _():
        denom = mx[0] + 1e-06
        ic = sc_ref[0]
        tsf = sc_ref[1]

        vm = vm_ref[...].reshape(ROWS, 128)
        wl = wl_ref[...].reshape(ROWS, 128)
        ls = ls_ref[...].reshape(ROWS, 128)

        v = TAU * vm + (1.0 - wl / denom)
        v = v + ic * (1.0 / (wl + 0.1))
        v = v + 0.1 * jnp.log1p(tsf - ls)

        v_out[...] = v.reshape(BLK)
        wl_out[...] = wl_ref[...]
        ls_out[...] = ls_ref[...]

        li = _local_iota()

        @pl.when(i < 2 * GRID - 1)
        def _():
            bmax = jnp.max(v)
            cand = jnp.min(jnp.where(v == bmax, li, jnp.int32(N)))
            pred = bmax > best[0]
            bidx[0] = jnp.where(pred, (i - GRID) * BLK + cand, bidx[0])
            best[0] = jnp.where(pred, bmax, best[0])

        @pl.when(i == 2 * GRID - 1)
        def _():
            masked = jnp.where(li < TAIL, v, -jnp.inf)
            bmax = jnp.max(masked)
            cand = jnp.min(jnp.where(masked == bmax, li, jnp.int32(N)))
            pred = bmax > best[0]
            widx_out[0] = jnp.where(pred, (i - GRID) * BLK + cand, bidx[0])


def _fix_body(w_ref, sc_ref, v_ref, wl_ref, ls_ref, vo_ref, wlo_ref, lso_ref):
    off = w_ref[0] % FBLK
    add = sc_ref[2]
    tsf = sc_ref[1]
    r = jax.lax.broadcasted_iota(jnp.int32, (FROWS, 128), 0)
    c = jax.lax.broadcasted_iota(jnp.int32, (FROWS, 128), 1)
    hit = (r * 128 + c) == off
    v = v_ref[...].reshape(FROWS, 128)
    wl = wl_ref[...].reshape(FROWS, 128)
    ls = ls_ref[...].reshape(FROWS, 128)
    vo_ref[...] = jnp.where(hit, 0.0, v).reshape(FBLK)
    wlo_ref[...] = jnp.where(hit, wl + add, wl).reshape(FBLK)
    lso_ref[...] = jnp.where(hit, tsf, ls).reshape(FBLK)


def kernel(v_mem, worker_load, last_spike, task_priority, task_complexity, timestep):
    f32 = jnp.float32
    tsf = f32(timestep)
    ic = task_priority * (1.0 + task_complexity)
    sc = jnp.stack([ic, tsf, task_complexity])

    wl_ix = lambda i: (jnp.where(i < GRID, i, i - GRID),)
    b_ix = lambda i: (jnp.maximum(i - GRID, 0),)
    smem = pl.BlockSpec(memory_space=pltpu.SMEM)
    v, wl_c, ls_c, widx = pl.pallas_call(
        _main_body,
        grid=(2 * GRID,),
        in_specs=[smem,
                  pl.BlockSpec((BLK,), b_ix),
                  pl.BlockSpec((BLK,), wl_ix),
                  pl.BlockSpec((BLK,), b_ix)],
        out_specs=[pl.BlockSpec((BLK,), b_ix)] * 3 + [smem],
        out_shape=[
            jax.ShapeDtypeStruct((N,), f32),
            jax.ShapeDtypeStruct((N,), f32),
            jax.ShapeDtypeStruct((N,), f32),
            jax.ShapeDtypeStruct((1,), jnp.int32),
        ],
        scratch_shapes=[pltpu.SMEM((1,), f32), pltpu.SMEM((1,), f32),
                        pltpu.SMEM((1,), jnp.int32)],
        compiler_params=pltpu.CompilerParams(
            dimension_semantics=("arbitrary",)),
    )(sc, v_mem, worker_load, last_spike)

    fblk = pl.BlockSpec((FBLK,), lambda i, w: (w[0] // FBLK,))
    grid_spec = pltpu.PrefetchScalarGridSpec(
        num_scalar_prefetch=1,
        grid=(1,),
        in_specs=[smem, fblk, fblk, fblk],
        out_specs=[fblk, fblk, fblk],
    )
    v_new, wl_new, ls_new = pl.pallas_call(
        _fix_body,
        grid_spec=grid_spec,
        out_shape=[jax.ShapeDtypeStruct((N,), f32)] * 3,
        input_output_aliases={2: 0, 3: 1, 4: 2},
    )(widx, sc, v, wl_c, ls_c)

    return widx[0], v_new, wl_new, ls_new
